# Initial kernel scaffold; baseline (speedup 1.0000x reference)
import jax
import jax.numpy as jnp
from jax.experimental import pallas as pl


def kernel(x, edge_index, adj_values, W1, b1, W2, b2, Wc, bc):
    raise NotImplementedError("write your pallas kernel here")


# speedup vs baseline: 5.1974x; 5.1974x over previous
import functools

import jax
import jax.numpy as jnp
from jax import lax
from jax.experimental import pallas as pl
from jax.experimental.pallas import tpu as pltpu
from jax.experimental.pallas import tpu_sc as plsc

_N = 10000
_E = 320000
_D = 128

_L = 16
_NC = 2
_NS = 16
_NW = _NC * _NS
_CHUNK = 128
_NCHUNKS = _E // _CHUNK
_ROWS_MAIN = 624


def _mm_bias_body(x_ref, w_ref, b_ref, o_ref):
    o_ref[...] = (
        jnp.dot(x_ref[...], w_ref[...], preferred_element_type=jnp.float32)
        + b_ref[...]
    )


def _mm_bias(x, w, b):
    return pl.pallas_call(
        _mm_bias_body,
        out_shape=jax.ShapeDtypeStruct((x.shape[0], w.shape[1]), jnp.float32),
    )(x, w, b.reshape(1, -1))


def _sum_mm_body(p_ref, w_ref, b_ref, o_ref, *, relu):
    h = p_ref[0] + p_ref[1]
    if relu:
        h = jnp.maximum(h, 0.0)
    o_ref[...] = (
        jnp.dot(h, w_ref[...], preferred_element_type=jnp.float32) + b_ref[...]
    )


def _sum_mm(p, w, b, relu):
    return pl.pallas_call(
        functools.partial(_sum_mm_body, relu=relu),
        out_shape=jax.ShapeDtypeStruct((p.shape[1], w.shape[1]), jnp.float32),
    )(p, w, b.reshape(1, -1))


def _spmm_sc(h, row, col, vals):
    mesh = plsc.VectorSubcoreMesh(core_axis_name="c", subcore_axis_name="s")

    @functools.partial(
        pl.kernel,
        mesh=mesh,
        out_type=jax.ShapeDtypeStruct((_NC, _N, _D), jnp.float32),
        scratch_types=[
            pltpu.VMEM((_CHUNK,), jnp.int32),
            pltpu.VMEM((_CHUNK,), jnp.int32),
            pltpu.VMEM((_CHUNK,), jnp.float32),
            pltpu.VMEM((_CHUNK, _D), jnp.float32),
            pltpu.VMEM((_CHUNK, _D), jnp.float32),
            pltpu.VMEM_SHARED((_N, _D), jnp.float32),
            pltpu.SemaphoreType.DMA,
        ],
    )
    def body(h_hbm, row_hbm, col_hbm, vals_hbm, out_hbm,
             col_v, row_v, vals_v, rows_v, zero_v, accum, sem):
        cid = lax.axis_index("c")
        sid = lax.axis_index("s")
        wid = sid * _NC + cid

        def zrow(r, carry):
            for j in range(_D // _L):
                zero_v[r, pl.ds(j * _L, _L)] = jnp.zeros((_L,), jnp.float32)
            return carry
        lax.fori_loop(0, _CHUNK, zrow, None)

        base0 = sid * _ROWS_MAIN
        for off, nn in ((0, 128), (128, 128), (256, 128), (384, 128), (512, 112)):
            pltpu.sync_copy(zero_v.at[pl.ds(0, nn), :],
                            accum.at[pl.ds(base0 + off, nn), :])

        @pl.when(sid == _NS - 1)
        def _zero_tail():
            pltpu.sync_copy(zero_v.at[pl.ds(0, 16), :],
                            accum.at[pl.ds(_ROWS_MAIN * _NS, 16), :])
        plsc.subcore_barrier()

        n_i = (_NCHUNKS - wid + _NW - 1) // _NW

        def chunk_body(i, carry):
            b = (wid + i * _NW) * _CHUNK
            pltpu.sync_copy(col_hbm.at[pl.ds(b, _CHUNK)], col_v)
            pltpu.sync_copy(vals_hbm.at[pl.ds(b, _CHUNK)], vals_v)
            pltpu.sync_copy(row_hbm.at[pl.ds(b, _CHUNK)], row_v)
            pltpu.async_copy(h_hbm.at[col_v], rows_v, sem).wait()

            def scale16(k16, c2):
                vals16 = vals_v[pl.ds(k16 * _L, _L)]
                for kk in range(_L):
                    val = lax.gather(
                        vals16,
                        jnp.full((_L, 1), kk, jnp.int32),
                        lax.GatherDimensionNumbers(
                            offset_dims=(), collapsed_slice_dims=(0,),
                            start_index_map=(0,)),
                        slice_sizes=(1,),
                        mode=lax.GatherScatterMode.PROMISE_IN_BOUNDS)
                    k = k16 * _L + kk
                    for j in range(_D // _L):
                        rows_v[k, pl.ds(j * _L, _L)] = (
                            rows_v[k, pl.ds(j * _L, _L)] * val
                        )
                return c2
            lax.fori_loop(0, _CHUNK // _L, scale16, None)

            pltpu.sync_copy(rows_v, accum.at[row_v], add=True)
            return carry
        lax.fori_loop(0, n_i, chunk_body, None)
        plsc.subcore_barrier()

        for off, nn in ((0, 128), (128, 128), (256, 128), (384, 128), (512, 112)):
            pltpu.sync_copy(accum.at[pl.ds(base0 + off, nn), :],
                            out_hbm.at[cid, pl.ds(base0 + off, nn), :])

        @pl.when(sid == _NS - 1)
        def _write_tail():
            pltpu.sync_copy(accum.at[pl.ds(_ROWS_MAIN * _NS, 16), :],
                            out_hbm.at[cid, pl.ds(_ROWS_MAIN * _NS, 16), :])

    return body(h, row, col, vals)


def kernel(x, edge_index, adj_values, W1, b1, W2, b2, Wc, bc):
    row = edge_index[0].astype(jnp.int32)
    col = edge_index[1].astype(jnp.int32)
    vals = adj_values.astype(jnp.float32)

    s1 = _mm_bias(x, W1, b1)
    p1 = _spmm_sc(s1, row, col, vals)
    s2 = _sum_mm(p1, W2, b2, relu=True)
    p2 = _spmm_sc(s2, row, col, vals)
    return _sum_mm(p2, Wc, bc, relu=False)


# speedup vs baseline: 8.8557x; 1.7039x over previous
import functools

import jax
import jax.numpy as jnp
from jax import lax
from jax.experimental import pallas as pl
from jax.experimental.pallas import tpu as pltpu
from jax.experimental.pallas import tpu_sc as plsc

_N = 10000
_E = 320000
_D = 128

_L = 16
_NC = 2
_NS = 16
_NW = _NC * _NS
_CHUNK = 128
_CPW = 78
_TAILC = _NW * _CPW
_ROWS_MAIN = 624

_GDN = lax.GatherDimensionNumbers(
    offset_dims=(), collapsed_slice_dims=(0,), start_index_map=(0,))


def _mm_bias_body(x_ref, w_ref, b_ref, o_ref):
    o_ref[...] = (
        jnp.dot(x_ref[...], w_ref[...], preferred_element_type=jnp.float32)
        + b_ref[...]
    )


def _mm_bias(x, w, b):
    return pl.pallas_call(
        _mm_bias_body,
        out_shape=jax.ShapeDtypeStruct((x.shape[0], w.shape[1]), jnp.float32),
    )(x, w, b.reshape(1, -1))


def _sum_mm_body(p_ref, w_ref, b_ref, o_ref, *, relu):
    h = p_ref[0] + p_ref[1]
    if relu:
        h = jnp.maximum(h, 0.0)
    o_ref[...] = (
        jnp.dot(h, w_ref[...], preferred_element_type=jnp.float32) + b_ref[...]
    )


def _sum_mm(p, w, b, relu):
    return pl.pallas_call(
        functools.partial(_sum_mm_body, relu=relu),
        out_shape=jax.ShapeDtypeStruct((p.shape[1], w.shape[1]), jnp.float32),
    )(p, w, b.reshape(1, -1))


def _spmm_sc(h, row, col, vals):
    mesh = plsc.VectorSubcoreMesh(core_axis_name="c", subcore_axis_name="s")

    @functools.partial(
        pl.kernel,
        mesh=mesh,
        out_type=jax.ShapeDtypeStruct((_NC, _N, _D), jnp.float32),
        scratch_types=[
            pltpu.VMEM((_CHUNK,), jnp.int32),
            pltpu.VMEM((_CHUNK,), jnp.int32),
            pltpu.VMEM((_CHUNK,), jnp.float32),
            pltpu.VMEM((_CHUNK,), jnp.int32),
            pltpu.VMEM((_CHUNK,), jnp.int32),
            pltpu.VMEM((_CHUNK,), jnp.float32),
            pltpu.VMEM((_CHUNK, _D), jnp.float32),
            pltpu.VMEM((_CHUNK, _D), jnp.float32),
            pltpu.VMEM_SHARED((_N, _D), jnp.float32),
            pltpu.SemaphoreType.DMA,
            pltpu.SemaphoreType.DMA,
            pltpu.SemaphoreType.DMA,
        ],
    )
    def body(h_hbm, row_hbm, col_hbm, vals_hbm, out_hbm,
             c0, r0, v0, c1, r1, v1, rows0, rows1, accum,
             isem, gsem0, gsem1):
        colb = (c0, c1)
        rowb = (r0, r1)
        valb = (v0, v1)
        rows = (rows0, rows1)
        gsem = (gsem0, gsem1)
        cid = lax.axis_index("c")
        sid = lax.axis_index("s")
        wid = sid * _NC + cid
        cbase = wid * _CPW

        def zrow(r, carry):
            for j in range(_D // _L):
                rows0[r, pl.ds(j * _L, _L)] = jnp.zeros((_L,), jnp.float32)
            return carry
        lax.fori_loop(0, _CHUNK, zrow, None)

        base0 = sid * _ROWS_MAIN
        for off, nn in ((0, 128), (128, 128), (256, 128), (384, 128), (512, 112)):
            pltpu.sync_copy(rows0.at[pl.ds(0, nn), :],
                            accum.at[pl.ds(base0 + off, nn), :])

        @pl.when(sid == _NS - 1)
        def _zero_tail():
            pltpu.sync_copy(rows0.at[pl.ds(0, 16), :],
                            accum.at[pl.ds(_ROWS_MAIN * _NS, 16), :])
        plsc.subcore_barrier()

        def idx_sync(g, s):
            b = g * _CHUNK
            pltpu.sync_copy(col_hbm.at[pl.ds(b, _CHUNK)], colb[s])
            pltpu.sync_copy(row_hbm.at[pl.ds(b, _CHUNK)], rowb[s])
            pltpu.sync_copy(vals_hbm.at[pl.ds(b, _CHUNK)], valb[s])

        def idx_start(g, s):
            b = g * _CHUNK
            pltpu.async_copy(col_hbm.at[pl.ds(b, _CHUNK)], colb[s], isem)
            pltpu.async_copy(row_hbm.at[pl.ds(b, _CHUNK)], rowb[s], isem)
            pltpu.async_copy(vals_hbm.at[pl.ds(b, _CHUNK)], valb[s], isem)

        def idx_wait(s):
            pltpu.make_async_copy(col_hbm.at[pl.ds(0, _CHUNK)],
                                  colb[s], isem).wait()
            pltpu.make_async_copy(row_hbm.at[pl.ds(0, _CHUNK)],
                                  rowb[s], isem).wait()
            pltpu.make_async_copy(vals_hbm.at[pl.ds(0, _CHUNK)],
                                  valb[s], isem).wait()

        def g_start(s):
            pltpu.async_copy(h_hbm.at[colb[s]], rows[s], gsem[s])

        def g_wait(s):
            pltpu.make_async_copy(h_hbm.at[colb[s]], rows[s],
                                  gsem[s]).wait()

        def scale(s):
            rr = rows[s]
            vv = valb[s]

            def s16(k16, carry):
                v16 = vv[pl.ds(k16 * _L, _L)]
                for kk in range(_L):
                    val = lax.gather(
                        v16, jnp.full((_L, 1), kk, jnp.int32), _GDN,
                        slice_sizes=(1,),
                        mode=lax.GatherScatterMode.PROMISE_IN_BOUNDS)
                    k = k16 * _L + kk
                    for jj in range(_D // _L):
                        rr[k, pl.ds(jj * _L, _L)] = (
                            rr[k, pl.ds(jj * _L, _L)] * val
                        )
                return carry
            lax.fori_loop(0, _CHUNK // _L, s16, None)

        def scat_sync(s):
            pltpu.sync_copy(rows[s], accum.at[rowb[s]], add=True)

        idx_sync(cbase + 0, 0)
        idx_sync(cbase + 1, 1)
        g_start(0)

        def pair(i, carry):
            c = cbase + 2 * i
            g_start(1)
            g_wait(0)
            scale(0)
            scat_sync(0)

            @pl.when(i < _CPW // 2 - 1)
            def _pf0():
                idx_start(c + 2, 0)
            g_wait(1)
            scale(1)
            scat_sync(1)

            @pl.when(i < _CPW // 2 - 1)
            def _pf1():
                idx_start(c + 3, 1)
                idx_wait(0)
                g_start(0)
                idx_wait(1)
            return carry
        lax.fori_loop(0, _CPW // 2, pair, None)

        @pl.when(wid < 4)
        def _tail():
            idx_sync(_TAILC + wid, 0)
            g_start(0)
            g_wait(0)
            scale(0)
            scat_sync(0)

        plsc.subcore_barrier()

        for off, nn in ((0, 128), (128, 128), (256, 128), (384, 128), (512, 112)):
            pltpu.sync_copy(accum.at[pl.ds(base0 + off, nn), :],
                            out_hbm.at[cid, pl.ds(base0 + off, nn), :])

        @pl.when(sid == _NS - 1)
        def _write_tail():
            pltpu.sync_copy(accum.at[pl.ds(_ROWS_MAIN * _NS, 16), :],
                            out_hbm.at[cid, pl.ds(_ROWS_MAIN * _NS, 16), :])

    return body(h, row, col, vals)


def kernel(x, edge_index, adj_values, W1, b1, W2, b2, Wc, bc):
    row = edge_index[0].astype(jnp.int32)
    col = edge_index[1].astype(jnp.int32)
    vals = adj_values.astype(jnp.float32)

    s1 = _mm_bias(x, W1, b1)
    p1 = _spmm_sc(s1, row, col, vals)
    s2 = _sum_mm(p1, W2, b2, relu=True)
    p2 = _spmm_sc(s2, row, col, vals)
    return _sum_mm(p2, Wc, bc, relu=False)


# speedup vs baseline: 10.3295x; 1.1664x over previous
import functools

import jax
import jax.numpy as jnp
from jax import lax
from jax.experimental import pallas as pl
from jax.experimental.pallas import tpu as pltpu
from jax.experimental.pallas import tpu_sc as plsc

_N = 10000
_E = 320000
_D = 128

_L = 16
_NC = 2
_NS = 16
_NW = _NC * _NS
_CHUNK = 128
_CPW = 78
_TAILC = _NW * _CPW
_ROWS_MAIN = 624

_GDN = lax.GatherDimensionNumbers(
    offset_dims=(), collapsed_slice_dims=(0,), start_index_map=(0,))


def _mm_bias_body(x_ref, w_ref, b_ref, o_ref):
    o_ref[...] = (
        jnp.dot(x_ref[...], w_ref[...], preferred_element_type=jnp.float32)
        + b_ref[...]
    )


def _mm_bias(x, w, b):
    return pl.pallas_call(
        _mm_bias_body,
        out_shape=jax.ShapeDtypeStruct((x.shape[0], w.shape[1]), jnp.float32),
    )(x, w, b.reshape(1, -1))


def _sum_mm_body(p_ref, w_ref, b_ref, o_ref, *, relu):
    h = p_ref[0] + p_ref[1]
    if relu:
        h = jnp.maximum(h, 0.0)
    o_ref[...] = (
        jnp.dot(h, w_ref[...], preferred_element_type=jnp.float32) + b_ref[...]
    )


def _sum_mm(p, w, b, relu):
    return pl.pallas_call(
        functools.partial(_sum_mm_body, relu=relu),
        out_shape=jax.ShapeDtypeStruct((p.shape[1], w.shape[1]), jnp.float32),
    )(p, w, b.reshape(1, -1))


def _spmm_sc(h, row, col, vals):
    mesh = plsc.VectorSubcoreMesh(core_axis_name="c", subcore_axis_name="s")

    @functools.partial(
        pl.kernel,
        mesh=mesh,
        out_type=jax.ShapeDtypeStruct((_NC, _N, _D), jnp.float32),
        scratch_types=[
            pltpu.VMEM((_CHUNK,), jnp.int32),
            pltpu.VMEM((_CHUNK,), jnp.int32),
            pltpu.VMEM((_CHUNK,), jnp.float32),
            pltpu.VMEM((_CHUNK,), jnp.int32),
            pltpu.VMEM((_CHUNK,), jnp.int32),
            pltpu.VMEM((_CHUNK,), jnp.float32),
            pltpu.VMEM((_CHUNK, _D), jnp.float32),
            pltpu.VMEM((_CHUNK, _D), jnp.float32),
            pltpu.VMEM((_CHUNK,), jnp.int32),
            pltpu.VMEM((_CHUNK,), jnp.int32),
            pltpu.VMEM_SHARED((_N, _D), jnp.float32),
            pltpu.SemaphoreType.DMA,
            pltpu.SemaphoreType.DMA,
            pltpu.SemaphoreType.DMA,
            pltpu.SemaphoreType.DMA,
            pltpu.SemaphoreType.DMA,
        ],
    )
    def body(h_hbm, row_hbm, col_hbm, vals_hbm, out_hbm,
             c0, r0, v0, c1, r1, v1, rows0, rows1, si0, si1, accum,
             isem, gsem0, gsem1, ssem0, ssem1):
        colb = (c0, c1)
        rowb = (r0, r1)
        valb = (v0, v1)
        rows = (rows0, rows1)
        sidx = (si0, si1)
        gsem = (gsem0, gsem1)
        ssem = (ssem0, ssem1)
        cid = lax.axis_index("c")
        sid = lax.axis_index("s")
        wid = sid * _NC + cid
        cbase = wid * _CPW

        def zrow(r, carry):
            for j in range(_D // _L):
                rows0[r, pl.ds(j * _L, _L)] = jnp.zeros((_L,), jnp.float32)
            return carry
        lax.fori_loop(0, _CHUNK, zrow, None)

        base0 = sid * _ROWS_MAIN
        for off, nn in ((0, 128), (128, 128), (256, 128), (384, 128), (512, 112)):
            pltpu.sync_copy(rows0.at[pl.ds(0, nn), :],
                            accum.at[pl.ds(base0 + off, nn), :])

        @pl.when(sid == _NS - 1)
        def _zero_tail():
            pltpu.sync_copy(rows0.at[pl.ds(0, 16), :],
                            accum.at[pl.ds(_ROWS_MAIN * _NS, 16), :])
        plsc.subcore_barrier()

        def idx_sync(g, s):
            b = g * _CHUNK
            pltpu.sync_copy(col_hbm.at[pl.ds(b, _CHUNK)], colb[s])
            pltpu.sync_copy(row_hbm.at[pl.ds(b, _CHUNK)], rowb[s])
            pltpu.sync_copy(vals_hbm.at[pl.ds(b, _CHUNK)], valb[s])

        def idx_start(g, s):
            b = g * _CHUNK
            pltpu.async_copy(col_hbm.at[pl.ds(b, _CHUNK)], colb[s], isem)
            pltpu.async_copy(row_hbm.at[pl.ds(b, _CHUNK)], rowb[s], isem)
            pltpu.async_copy(vals_hbm.at[pl.ds(b, _CHUNK)], valb[s], isem)

        def idx_wait(s):
            pltpu.make_async_copy(col_hbm.at[pl.ds(0, _CHUNK)],
                                  colb[s], isem).wait()
            pltpu.make_async_copy(row_hbm.at[pl.ds(0, _CHUNK)],
                                  rowb[s], isem).wait()
            pltpu.make_async_copy(vals_hbm.at[pl.ds(0, _CHUNK)],
                                  valb[s], isem).wait()

        def g_start(s):
            pltpu.async_copy(h_hbm.at[colb[s]], rows[s], gsem[s])

        def g_wait(s):
            pltpu.make_async_copy(h_hbm.at[colb[s]], rows[s],
                                  gsem[s]).wait()

        def scale(s):
            rr = rows[s]
            vv = valb[s]

            def s16(k16, carry):
                v16 = vv[pl.ds(k16 * _L, _L)]
                for kk in range(_L):
                    val = lax.gather(
                        v16, jnp.full((_L, 1), kk, jnp.int32), _GDN,
                        slice_sizes=(1,),
                        mode=lax.GatherScatterMode.PROMISE_IN_BOUNDS)
                    k = k16 * _L + kk
                    for jj in range(_D // _L):
                        rr[k, pl.ds(jj * _L, _L)] = (
                            rr[k, pl.ds(jj * _L, _L)] * val
                        )
                return carry
            lax.fori_loop(0, _CHUNK // _L, s16, None)

        def scat_sync(s):
            pltpu.sync_copy(rows[s], accum.at[rowb[s]], add=True)

        def sc_start(s):
            for i in range(_CHUNK // _L):
                sidx[s][pl.ds(i * _L, _L)] = rowb[s][pl.ds(i * _L, _L)]
            pltpu.async_copy(rows[s], accum.at[sidx[s]], ssem[s], add=True)

        def sc_wait(s):
            pltpu.make_async_copy(rows[s], accum.at[sidx[s]],
                                  ssem[s]).wait()

        idx_sync(cbase + 0, 0)
        idx_sync(cbase + 1, 1)
        g_start(0)

        def pair(i, carry):
            c = cbase + 2 * i

            @pl.when(i > 0)
            def _drain1():
                sc_wait(1)
            g_start(1)
            g_wait(0)
            scale(0)
            sc_start(0)

            @pl.when(i < _CPW // 2 - 1)
            def _pf0():
                idx_start(c + 2, 0)
            g_wait(1)
            scale(1)
            sc_start(1)

            @pl.when(i < _CPW // 2 - 1)
            def _pf1():
                idx_start(c + 3, 1)
                idx_wait(0)
                sc_wait(0)
                g_start(0)
                idx_wait(1)
            return carry
        lax.fori_loop(0, _CPW // 2, pair, None)
        sc_wait(0)
        sc_wait(1)

        @pl.when(wid < 4)
        def _tail():
            idx_sync(_TAILC + wid, 0)
            g_start(0)
            g_wait(0)
            scale(0)
            scat_sync(0)

        plsc.subcore_barrier()

        for off, nn in ((0, 128), (128, 128), (256, 128), (384, 128), (512, 112)):
            pltpu.sync_copy(accum.at[pl.ds(base0 + off, nn), :],
                            out_hbm.at[cid, pl.ds(base0 + off, nn), :])

        @pl.when(sid == _NS - 1)
        def _write_tail():
            pltpu.sync_copy(accum.at[pl.ds(_ROWS_MAIN * _NS, 16), :],
                            out_hbm.at[cid, pl.ds(_ROWS_MAIN * _NS, 16), :])

    return body(h, row, col, vals)


def kernel(x, edge_index, adj_values, W1, b1, W2, b2, Wc, bc):
    row = edge_index[0].astype(jnp.int32)
    col = edge_index[1].astype(jnp.int32)
    vals = adj_values.astype(jnp.float32)

    s1 = _mm_bias(x, W1, b1)
    p1 = _spmm_sc(s1, row, col, vals)
    s2 = _sum_mm(p1, W2, b2, relu=True)
    p2 = _spmm_sc(s2, row, col, vals)
    return _sum_mm(p2, Wc, bc, relu=False)


# speedup vs baseline: 12.3354x; 1.1942x over previous
import functools

import jax
import jax.numpy as jnp
from jax import lax
from jax.experimental import pallas as pl
from jax.experimental.pallas import tpu as pltpu
from jax.experimental.pallas import tpu_sc as plsc

_N = 10000
_E = 320000
_D = 128

_L = 16
_NC = 2
_NS = 16
_NW = _NC * _NS
_CHUNK = 128
_CPW = 78
_TAILC = _NW * _CPW
_ROWS_MAIN = 624

_GDN = lax.GatherDimensionNumbers(
    offset_dims=(), collapsed_slice_dims=(0,), start_index_map=(0,))


def _mm_bias_body(x_ref, w_ref, b_ref, o_ref):
    o_ref[...] = (
        jnp.dot(x_ref[...], w_ref[...], preferred_element_type=jnp.float32)
        + b_ref[...]
    )


def _mm_bias(x, w, b):
    return pl.pallas_call(
        _mm_bias_body,
        out_shape=jax.ShapeDtypeStruct((x.shape[0], w.shape[1]), jnp.float32),
    )(x, w, b.reshape(1, -1))


def _sum_mm_body(p_ref, w_ref, b_ref, o_ref, *, relu):
    h = p_ref[0] + p_ref[1]
    if relu:
        h = jnp.maximum(h, 0.0)
    o_ref[...] = (
        jnp.dot(h, w_ref[...], preferred_element_type=jnp.float32) + b_ref[...]
    )


def _sum_mm(p, w, b, relu):
    return pl.pallas_call(
        functools.partial(_sum_mm_body, relu=relu),
        out_shape=jax.ShapeDtypeStruct((p.shape[1], w.shape[1]), jnp.float32),
    )(p, w, b.reshape(1, -1))


def _spmm_sc(h, row, col, vals):
    mesh = plsc.VectorSubcoreMesh(core_axis_name="c", subcore_axis_name="s")

    @functools.partial(
        pl.kernel,
        mesh=mesh,
        out_type=jax.ShapeDtypeStruct((_NC, _N, _D), jnp.float32),
        scratch_types=[
            pltpu.VMEM((_CHUNK,), jnp.int32),
            pltpu.VMEM((_CHUNK,), jnp.int32),
            pltpu.VMEM((_CHUNK,), jnp.float32),
            pltpu.VMEM((_CHUNK,), jnp.int32),
            pltpu.VMEM((_CHUNK,), jnp.int32),
            pltpu.VMEM((_CHUNK,), jnp.float32),
            pltpu.VMEM((_CHUNK,), jnp.int32),
            pltpu.VMEM((_CHUNK,), jnp.int32),
            pltpu.VMEM((_CHUNK,), jnp.float32),
            pltpu.VMEM((_CHUNK, _D), jnp.float32),
            pltpu.VMEM((_CHUNK, _D), jnp.float32),
            pltpu.VMEM((_CHUNK, _D), jnp.float32),
            pltpu.VMEM((_CHUNK,), jnp.int32),
            pltpu.VMEM((_CHUNK,), jnp.int32),
            pltpu.VMEM((_CHUNK,), jnp.int32),
            pltpu.VMEM_SHARED((_N, _D), jnp.float32),
            pltpu.SemaphoreType.DMA,
            pltpu.SemaphoreType.DMA,
            pltpu.SemaphoreType.DMA,
            pltpu.SemaphoreType.DMA,
            pltpu.SemaphoreType.DMA,
            pltpu.SemaphoreType.DMA,
            pltpu.SemaphoreType.DMA,
        ],
    )
    def body(h_hbm, row_hbm, col_hbm, vals_hbm, out_hbm,
             c0, r0, v0, c1, r1, v1, c2, r2, v2,
             rows0, rows1, rows2, si0, si1, si2, accum,
             isem, gsem0, gsem1, gsem2, ssem0, ssem1, ssem2):
        colb = (c0, c1, c2)
        rowb = (r0, r1, r2)
        valb = (v0, v1, v2)
        rows = (rows0, rows1, rows2)
        sidx = (si0, si1, si2)
        gsem = (gsem0, gsem1, gsem2)
        ssem = (ssem0, ssem1, ssem2)
        cid = lax.axis_index("c")
        sid = lax.axis_index("s")
        wid = sid * _NC + cid
        cbase = wid * _CPW

        def zrow(r, carry):
            for j in range(_D // _L):
                rows0[r, pl.ds(j * _L, _L)] = jnp.zeros((_L,), jnp.float32)
            return carry
        lax.fori_loop(0, _CHUNK, zrow, None)

        base0 = sid * _ROWS_MAIN
        for off, nn in ((0, 128), (128, 128), (256, 128), (384, 128), (512, 112)):
            pltpu.sync_copy(rows0.at[pl.ds(0, nn), :],
                            accum.at[pl.ds(base0 + off, nn), :])

        @pl.when(sid == _NS - 1)
        def _zero_tail():
            pltpu.sync_copy(rows0.at[pl.ds(0, 16), :],
                            accum.at[pl.ds(_ROWS_MAIN * _NS, 16), :])
        plsc.subcore_barrier()

        def idx_sync(g, s):
            b = g * _CHUNK
            pltpu.sync_copy(col_hbm.at[pl.ds(b, _CHUNK)], colb[s])
            pltpu.sync_copy(row_hbm.at[pl.ds(b, _CHUNK)], rowb[s])
            pltpu.sync_copy(vals_hbm.at[pl.ds(b, _CHUNK)], valb[s])

        def idx_start(g, s):
            b = g * _CHUNK
            pltpu.async_copy(col_hbm.at[pl.ds(b, _CHUNK)], colb[s], isem)
            pltpu.async_copy(row_hbm.at[pl.ds(b, _CHUNK)], rowb[s], isem)
            pltpu.async_copy(vals_hbm.at[pl.ds(b, _CHUNK)], valb[s], isem)

        def idx_wait(s):
            pltpu.make_async_copy(col_hbm.at[pl.ds(0, _CHUNK)],
                                  colb[s], isem).wait()
            pltpu.make_async_copy(row_hbm.at[pl.ds(0, _CHUNK)],
                                  rowb[s], isem).wait()
            pltpu.make_async_copy(vals_hbm.at[pl.ds(0, _CHUNK)],
                                  valb[s], isem).wait()

        def g_start(s):
            pltpu.async_copy(h_hbm.at[colb[s]], rows[s], gsem[s])

        def g_wait(s):
            pltpu.make_async_copy(h_hbm.at[colb[s]], rows[s],
                                  gsem[s]).wait()

        def scale(s):
            rr = rows[s]
            vv = valb[s]

            def s16(k16, carry):
                v16 = vv[pl.ds(k16 * _L, _L)]
                for kk in range(_L):
                    val = lax.gather(
                        v16, jnp.full((_L, 1), kk, jnp.int32), _GDN,
                        slice_sizes=(1,),
                        mode=lax.GatherScatterMode.PROMISE_IN_BOUNDS)
                    k = k16 * _L + kk
                    for jj in range(_D // _L):
                        rr[k, pl.ds(jj * _L, _L)] = (
                            rr[k, pl.ds(jj * _L, _L)] * val
                        )
                return carry
            lax.fori_loop(0, _CHUNK // _L, s16, None)

        def scat_sync(s):
            pltpu.sync_copy(rows[s], accum.at[rowb[s]], add=True)

        def sc_start(s):
            for i in range(_CHUNK // _L):
                sidx[s][pl.ds(i * _L, _L)] = rowb[s][pl.ds(i * _L, _L)]
            pltpu.async_copy(rows[s], accum.at[sidx[s]], ssem[s], add=True)

        def sc_wait(s):
            pltpu.make_async_copy(rows[s], accum.at[sidx[s]],
                                  ssem[s]).wait()

        _NT = _CPW // 3
        idx_sync(cbase + 0, 0)
        idx_sync(cbase + 1, 1)
        idx_start(cbase + 2, 2)
        g_start(0)

        def triple(t, carry):
            c = cbase + 3 * t

            @pl.when(t > 0)
            def _d1():
                sc_wait(1)
                idx_wait(1)
            g_start(1)
            g_wait(0)
            scale(0)
            sc_start(0)

            @pl.when(t < _NT - 1)
            def _pf0():
                idx_start(c + 3, 0)

            @pl.when(t > 0)
            def _d2():
                sc_wait(2)
            idx_wait(2)
            g_start(2)
            g_wait(1)
            scale(1)
            sc_start(1)

            @pl.when(t < _NT - 1)
            def _pf1():
                idx_start(c + 4, 1)

            @pl.when(t < _NT - 1)
            def _n0():
                sc_wait(0)
                idx_wait(0)
                g_start(0)
            g_wait(2)
            scale(2)
            sc_start(2)

            @pl.when(t < _NT - 1)
            def _pf2():
                idx_start(c + 5, 2)
            return carry
        lax.fori_loop(0, _NT, triple, None)
        sc_wait(0)
        sc_wait(1)
        sc_wait(2)

        @pl.when(wid < 4)
        def _tail():
            idx_sync(_TAILC + wid, 0)
            g_start(0)
            g_wait(0)
            scale(0)
            scat_sync(0)

        plsc.subcore_barrier()

        for off, nn in ((0, 128), (128, 128), (256, 128), (384, 128), (512, 112)):
            pltpu.sync_copy(accum.at[pl.ds(base0 + off, nn), :],
                            out_hbm.at[cid, pl.ds(base0 + off, nn), :])

        @pl.when(sid == _NS - 1)
        def _write_tail():
            pltpu.sync_copy(accum.at[pl.ds(_ROWS_MAIN * _NS, 16), :],
                            out_hbm.at[cid, pl.ds(_ROWS_MAIN * _NS, 16), :])

    return body(h, row, col, vals)


def kernel(x, edge_index, adj_values, W1, b1, W2, b2, Wc, bc):
    row = edge_index[0].astype(jnp.int32)
    col = edge_index[1].astype(jnp.int32)
    vals = adj_values.astype(jnp.float32)

    s1 = _mm_bias(x, W1, b1)
    p1 = _spmm_sc(s1, row, col, vals)
    s2 = _sum_mm(p1, W2, b2, relu=True)
    p2 = _spmm_sc(s2, row, col, vals)
    return _sum_mm(p2, Wc, bc, relu=False)
